# baseline (speedup 1.0000x reference)
import functools

import jax
import jax.numpy as jnp
import numpy as np
from jax.experimental import pallas as pl

D = 128
K = 17
N_LAYERS = 3


def _knn_idx(x, y, k):
    C = 1000
    N = x.shape[0]

    def chunk_fn(xq):
        d = jnp.sum((xq[:, None, :] - y[None, :, :]) ** 2, axis=-1)
        _, idx = jax.lax.top_k(-d, k)
        return idx

    xs = x.reshape(N // C, C, x.shape[1])
    idx = jax.lax.map(chunk_fn, xs)
    return idx.reshape(N, k)


def _leaky(v):
    return jnp.where(v >= 0, v, 0.2 * v)


def _group_norm(v, gamma, beta, groups=2, eps=1e-5):
    N, Dd = v.shape
    vg = v.reshape(N, groups, Dd // groups)
    mean = vg.mean(axis=-1, keepdims=True)
    var = vg.var(axis=-1, keepdims=True)
    vg = (vg - mean) / jnp.sqrt(var + eps)
    return vg.reshape(N, Dd) * gamma[None, :] + beta[None, :]


def _final_kernel(out_ref, msg_ref, gamma_ref, beta_ref, o_ref):
    v = msg_ref[...]
    groups = 2
    Nb, Dd = v.shape
    vg = v.reshape(Nb, groups, Dd // groups)
    mean = vg.mean(axis=-1, keepdims=True)
    var = vg.var(axis=-1, keepdims=True)
    vg = (vg - mean) / jnp.sqrt(var + 1e-5)
    v = vg.reshape(Nb, Dd) * gamma_ref[...] + beta_ref[...]
    v = jnp.where(v >= 0, v, 0.2 * v)
    o_ref[...] = out_ref[...] + v


def _residual_update(out, messages, gamma_i, beta_i):
    N = out.shape[0]
    B = 1000
    return pl.pallas_call(
        _final_kernel,
        out_shape=jax.ShapeDtypeStruct((N, D), jnp.float32),
        grid=(N // B,),
        in_specs=[
            pl.BlockSpec((B, D), lambda i: (i, 0)),
            pl.BlockSpec((B, D), lambda i: (i, 0)),
            pl.BlockSpec((1, D), lambda i: (0, 0)),
            pl.BlockSpec((1, D), lambda i: (0, 0)),
        ],
        out_specs=pl.BlockSpec((B, D), lambda i: (i, 0)),
    )(out, messages, gamma_i.reshape(1, D), beta_i.reshape(1, D))


def kernel(x, y, y_atomtypes, x_batch, y_batch, W1, b1, W2, b2, gamma, beta):
    idx_full = _knn_idx(x, y, K)
    idx = idx_full[:, 1:]
    k = K - 1
    dists = jnp.sum((x[:, None, :] - y[idx]) ** 2, axis=-1)
    out = y_atomtypes
    for i in range(N_LAYERS):
        A = out @ W1[i][:D] + b1[i]
        Bm = out @ W1[i][D:2 * D]
        w_d = W1[i][2 * D]
        h = _leaky(A[:, None, :] + Bm[idx] + dists[:, :, None] * w_d[None, None, :])
        S = h.sum(axis=1)
        messages = S @ W2[i] + k * b2[i]
        out = _residual_update(out, messages, gamma[i], beta[i])
    return out


# speedup vs baseline: 7.7813x; 7.7813x over previous
import functools

import jax
import jax.numpy as jnp
from jax.experimental import pallas as pl
from jax.experimental.pallas import tpu as pltpu
from jax.experimental.pallas import tpu_sc as plsc

D = 128
K = 17
N_LAYERS = 3

NP = 10240
NC = 640
CB = 128
QB = 2048
L_CH = 20
KSEL = 17
W = 32
RPW = NP // W
GPW = RPW // 16


def _cmin_body(x_ref, y_ref, o_ref):
    acc = jnp.full((QB, NC), 1e30, dtype=jnp.float32)
    for s in range(16):
        dx = x_ref[:, 0:1] - y_ref[3 * s:3 * s + 1, :]
        dy = x_ref[:, 1:2] - y_ref[3 * s + 1:3 * s + 2, :]
        dz = x_ref[:, 2:3] - y_ref[3 * s + 2:3 * s + 3, :]
        acc = jnp.minimum(acc, dx * dx + dy * dy + dz * dz)
    o_ref[...] = acc


def _cmin_tc(xp, y_t):
    return pl.pallas_call(
        _cmin_body,
        out_shape=jax.ShapeDtypeStruct((NP, NC), jnp.float32),
        grid=(NP // QB,),
        in_specs=[
            pl.BlockSpec((QB, 3), lambda q: (q, 0)),
            pl.BlockSpec((48, NC), lambda q: (0, 0)),
        ],
        out_specs=pl.BlockSpec((QB, NC), lambda q: (q, 0)),
    )(xp, y_t)


def _knn_sc_body(cmins, x0h, x1h, x2h, y0h, y1h, y2h, idx_out, dst_out,
                 tile, y0v, y1v, y2v, x0v, x1v, x2v, chs, obi, obd):
    wid = jax.lax.axis_index("s") * 2 + jax.lax.axis_index("c")
    pltpu.sync_copy(y0h, y0v)
    pltpu.sync_copy(y1h, y1v)
    pltpu.sync_copy(y2h, y2v)
    base_row = wid * RPW
    pltpu.sync_copy(x0h.at[pl.ds(base_row, RPW)], x0v)
    pltpu.sync_copy(x1h.at[pl.ds(base_row, RPW)], x1v)
    pltpu.sync_copy(x2h.at[pl.ds(base_row, RPW)], x2v)

    def group_body(g, _):
        r0 = base_row + g * 16
        pltpu.sync_copy(cmins.at[pl.ds(r0 * NC, 16 * NC)], tile)

        init = tuple(jnp.full((16,), 0x7FFFFFFF, dtype=jnp.int32)
                     for _ in range(L_CH))

        def sel_body(blk, carry):
            lst = list(carry)
            row_nc = jax.lax.iota(jnp.int32, 16) * NC
            for u in range(4):
                c = blk * 4 + u
                v = plsc.load_gather(tile, [row_nc + c])
                kb = jax.lax.bitcast_convert_type(v, jnp.int32)
                key = (kb & (-1024)) | c
                t = key
                for j in range(L_CH):
                    lo = jnp.minimum(lst[j], t)
                    hi = jnp.maximum(lst[j], t)
                    lst[j] = lo
                    t = hi
            return tuple(lst)

        lst = jax.lax.fori_loop(0, NC // 4, sel_body, init)
        for j in range(L_CH):
            chs[pl.ds(j * 16, 16)] = lst[j] & 1023

        g16 = g * 16
        xl0 = x0v[pl.ds(g16, 16)]
        xl1 = x1v[pl.ds(g16, 16)]
        xl2 = x2v[pl.ds(g16, 16)]
        dinit = tuple(jnp.full((16,), 1e30, dtype=jnp.float32)
                      for _ in range(KSEL))
        iinit = tuple(jnp.full((16,), 0, dtype=jnp.int32)
                      for _ in range(KSEL))

        def ex_body(ci, carry):
            dl = list(carry[0])
            il = list(carry[1])
            ch = chs[pl.ds(ci * 16, 16)]
            cbase = ch * 16
            for s in range(16):
                cand = cbase + s
                gy0 = plsc.load_gather(y0v, [cand])
                gy1 = plsc.load_gather(y1v, [cand])
                gy2 = plsc.load_gather(y2v, [cand])
                dx = xl0 - gy0
                dy = xl1 - gy1
                dz = xl2 - gy2
                td = dx * dx + dy * dy + dz * dz
                ti = cand
                for j in range(KSEL):
                    m = td < dl[j]
                    nd = jnp.where(m, td, dl[j])
                    hd = jnp.where(m, dl[j], td)
                    ni = jnp.where(m, ti, il[j])
                    hi = jnp.where(m, il[j], ti)
                    dl[j] = nd
                    td = hd
                    il[j] = ni
                    ti = hi
            return (tuple(dl), tuple(il))

        dl, il = jax.lax.fori_loop(0, L_CH, ex_body, (dinit, iinit))
        row16 = jax.lax.iota(jnp.int32, 16) * 16
        for j in range(16):
            plsc.store_scatter(obi, [row16 + j], il[j + 1])
            plsc.store_scatter(obd, [row16 + j], dl[j + 1])
        pltpu.sync_copy(obi, idx_out.at[pl.ds(r0 * 16, 256)])
        pltpu.sync_copy(obd, dst_out.at[pl.ds(r0 * 16, 256)])
        return 0

    jax.lax.fori_loop(0, GPW, group_body, 0)


def _knn(x, y):
    n = x.shape[0]
    xp = jnp.zeros((NP, 3), jnp.float32).at[:n].set(x)
    yp = jnp.full((NP, 3), 1e4, jnp.float32).at[:n].set(y)
    y_t = yp.reshape(NC, 48).T
    cmins = _cmin_tc(xp, y_t).reshape(NP * NC)

    mesh = plsc.VectorSubcoreMesh(core_axis_name="c", subcore_axis_name="s")
    f = functools.partial(
        pl.kernel,
        mesh=mesh,
        compiler_params=pltpu.CompilerParams(needs_layout_passes=False,
                                             use_tc_tiling_on_sc=False),
        out_type=[
            jax.ShapeDtypeStruct((NP * 16,), jnp.int32),
            jax.ShapeDtypeStruct((NP * 16,), jnp.float32),
        ],
        scratch_types=[
            pltpu.VMEM((16 * NC,), jnp.float32),
            pltpu.VMEM((NP,), jnp.float32),
            pltpu.VMEM((NP,), jnp.float32),
            pltpu.VMEM((NP,), jnp.float32),
            pltpu.VMEM((RPW,), jnp.float32),
            pltpu.VMEM((RPW,), jnp.float32),
            pltpu.VMEM((RPW,), jnp.float32),
            pltpu.VMEM((L_CH * 16,), jnp.int32),
            pltpu.VMEM((256,), jnp.int32),
            pltpu.VMEM((256,), jnp.float32),
        ],
    )(_knn_sc_body)
    idx_t, dst_t = f(cmins, xp[:, 0], xp[:, 1], xp[:, 2],
                     yp[:, 0], yp[:, 1], yp[:, 2])
    return idx_t, dst_t


HP = 272
HV = HP // 16
RB = 512


def _dense_body(x_ref, wa_ref, wb_ref, bias_ref, c_ref, b_ref):
    xb = x_ref[...]
    c_ref[...] = jnp.dot(xb, wa_ref[...],
                         preferred_element_type=jnp.float32) + bias_ref[...]
    b_ref[...] = jnp.dot(xb, wb_ref[...], preferred_element_type=jnp.float32)


def _dense_tc(outp, wa, wb, bias):
    return pl.pallas_call(
        _dense_body,
        out_shape=[jax.ShapeDtypeStruct((NP, HP), jnp.float32),
                   jax.ShapeDtypeStruct((NP, HP), jnp.float32)],
        grid=(NP // RB,),
        in_specs=[
            pl.BlockSpec((RB, D), lambda i: (i, 0)),
            pl.BlockSpec((D, HP), lambda i: (0, 0)),
            pl.BlockSpec((D, HP), lambda i: (0, 0)),
            pl.BlockSpec((1, HP), lambda i: (0, 0)),
        ],
        out_specs=[pl.BlockSpec((RB, HP), lambda i: (i, 0)),
                   pl.BlockSpec((RB, HP), lambda i: (i, 0))],
    )(outp, wa, wb, bias)


def _post_body(s_ref, w2_ref, b2_ref, gamma_ref, beta_ref, out_ref, gm_ref,
               o_ref):
    msg = jnp.dot(s_ref[...], w2_ref[...],
                  preferred_element_type=jnp.float32) + b2_ref[...]
    gm = gm_ref[...]
    mean = jnp.dot(msg, gm, preferred_element_type=jnp.float32)
    ex2 = jnp.dot(msg * msg, gm, preferred_element_type=jnp.float32)
    var = ex2 - mean * mean
    v = (msg - mean) / jnp.sqrt(var + 1e-5)
    v = v * gamma_ref[...] + beta_ref[...]
    v = jnp.where(v >= 0, v, 0.2 * v)
    o_ref[...] = out_ref[...] + v


def _post_tc(S, w2p, b2s, gamma_i, beta_i, outp, gmat):
    return pl.pallas_call(
        _post_body,
        out_shape=jax.ShapeDtypeStruct((NP, D), jnp.float32),
        grid=(NP // RB,),
        in_specs=[
            pl.BlockSpec((RB, HP), lambda i: (i, 0)),
            pl.BlockSpec((HP, D), lambda i: (0, 0)),
            pl.BlockSpec((1, D), lambda i: (0, 0)),
            pl.BlockSpec((1, D), lambda i: (0, 0)),
            pl.BlockSpec((1, D), lambda i: (0, 0)),
            pl.BlockSpec((RB, D), lambda i: (i, 0)),
            pl.BlockSpec((D, D), lambda i: (0, 0)),
        ],
        out_specs=pl.BlockSpec((RB, D), lambda i: (i, 0)),
    )(S, w2p, b2s, gamma_i, beta_i, outp, gmat)


def _edge_sc_body(c_flat, bm, wd_h, idx_h, dst_h, s_out,
                  idxv, dstv, wdv, cv, bufa, bufb, sbuf, bsema, bsemb):
    wid = jax.lax.axis_index("s") * 2 + jax.lax.axis_index("c")
    r0 = wid * RPW
    pltpu.sync_copy(idx_h.at[pl.ds(r0 * 16, RPW * 16)], idxv)
    pltpu.sync_copy(dst_h.at[pl.ds(r0 * 16, RPW * 16)], dstv)
    pltpu.sync_copy(wd_h, wdv)
    pltpu.sync_copy(c_flat.at[pl.ds(r0 * HP, RPW * HP)], cv)
    wd = [wdv[pl.ds(16 * v, 16)] for v in range(HV)]

    def issue(r, buf, sem):
        iv = idxv[pl.ds(r * 16, 16)]
        pltpu.async_copy(bm.at[iv], buf, sem)

    def wait(buf, sem):
        pltpu.make_async_copy(bm.at[idxv[pl.ds(0, 16)]], buf, sem).wait()

    def compute(r, buf):
        def s_body(sidx, acc):
            dsp = plsc.load_gather(dstv, [jnp.broadcast_to(r * 16 + sidx,
                                                           (16,))])
            new = []
            for v in range(HV):
                t = (cv[pl.ds(r * HP + 16 * v, 16)]
                     + buf[sidx, pl.ds(16 * v, 16)]
                     + dsp * wd[v])
                t = jnp.maximum(t, 0.2 * t)
                new.append(acc[v] + t)
            return tuple(new)

        acc = jax.lax.fori_loop(
            0, 16, s_body, tuple(jnp.zeros((16,), jnp.float32)
                                 for _ in range(HV)))
        sg = jax.lax.rem(r, 16)
        for v in range(HV):
            sbuf[pl.ds(sg * HP + 16 * v, 16)] = acc[v]

    issue(0, bufa, bsema)

    def row_body(r2, _):
        r = 2 * r2
        issue(r + 1, bufb, bsemb)
        wait(bufa, bsema)
        compute(r, bufa)

        @pl.when(r2 < RPW // 2 - 1)
        def _():
            issue(r + 2, bufa, bsema)

        wait(bufb, bsemb)
        compute(r + 1, bufb)

        @pl.when(jax.lax.rem(r2, 8) == 7)
        def _():
            base = (r2 - 7) * 2
            pltpu.sync_copy(sbuf,
                            s_out.at[pl.ds((r0 + base) * HP, 16 * HP)])

        return 0

    jax.lax.fori_loop(0, RPW // 2, row_body, 0)


def _edge_sc(c2d, bm, wd, idxf, dstf):
    mesh = plsc.VectorSubcoreMesh(core_axis_name="c", subcore_axis_name="s")
    f = functools.partial(
        pl.kernel,
        mesh=mesh,
        compiler_params=pltpu.CompilerParams(needs_layout_passes=False,
                                             use_tc_tiling_on_sc=False),
        out_type=jax.ShapeDtypeStruct((NP * HP,), jnp.float32),
        scratch_types=[
            pltpu.VMEM((RPW * 16,), jnp.int32),
            pltpu.VMEM((RPW * 16,), jnp.float32),
            pltpu.VMEM((HP,), jnp.float32),
            pltpu.VMEM((RPW * HP,), jnp.float32),
            pltpu.VMEM((16, HP), jnp.float32),
            pltpu.VMEM((16, HP), jnp.float32),
            pltpu.VMEM((16 * HP,), jnp.float32),
            pltpu.SemaphoreType.DMA,
            pltpu.SemaphoreType.DMA,
        ],
    )(_edge_sc_body)
    return f(c2d.reshape(NP * HP), bm, wd, idxf, dstf).reshape(NP, HP)


def kernel(x, y, y_atomtypes, x_batch, y_batch, W1, b1, W2, b2, gamma, beta):
    idxf, dstf = _knn(x, y)
    k = K - 1
    outp = jnp.zeros((NP, D), jnp.float32).at[:y_atomtypes.shape[0]].set(
        y_atomtypes)
    gmat = jnp.kron(jnp.eye(2, dtype=jnp.float32),
                    jnp.full((D // 2, D // 2), 1.0 / (D // 2), jnp.float32))
    for i in range(N_LAYERS):
        wa = jnp.concatenate([W1[i][:D], jnp.zeros((D, HP - (2 * D + 1)),
                                                   jnp.float32)], axis=1)
        wb = jnp.concatenate([W1[i][D:2 * D],
                              jnp.zeros((D, HP - (2 * D + 1)), jnp.float32)],
                             axis=1)
        bias = jnp.concatenate([b1[i], jnp.zeros((HP - (2 * D + 1),),
                                                 jnp.float32)]).reshape(1, HP)
        wd = jnp.concatenate([W1[i][2 * D], jnp.zeros((HP - (2 * D + 1),),
                                                      jnp.float32)])
        w2p = jnp.concatenate([W2[i], jnp.zeros((HP - (2 * D + 1), D),
                                                jnp.float32)], axis=0)
        b2s = (k * b2[i]).reshape(1, D)
        C, Bm = _dense_tc(outp, wa, wb, bias)
        S = _edge_sc(C, Bm, wd, idxf, dstf)
        outp = _post_tc(S, w2p, b2s, gamma[i].reshape(1, D),
                        beta[i].reshape(1, D), outp, gmat)
    return outp[:y_atomtypes.shape[0]]
